# baseline (speedup 1.0000x reference)
import functools
import jax
import jax.numpy as jnp
from jax import lax
from jax.experimental import pallas as pl
from jax.experimental.pallas import tpu as pltpu
from jax.experimental.pallas import tpu_sc as plsc

NC = 2
NS = 16
NW = NC * NS
L = 16

def _mesh():
  return plsc.VectorSubcoreMesh(core_axis_name="c", subcore_axis_name="s",
                                num_cores=NC, num_subcores=NS)


def _make_hist(E, N_pad):
  EPW = E // NW
  CHUNK = 2000
  NCHUNK = EPW // CHUNK
  N = N_pad

  @functools.partial(
      pl.kernel,
      out_type=jax.ShapeDtypeStruct((NW, N), jnp.float32),
      mesh=_mesh(),
      compiler_params=pltpu.CompilerParams(needs_layout_passes=False),
      scratch_types=[
          pltpu.VMEM((N,), jnp.float32),
          pltpu.VMEM((CHUNK,), jnp.int32),
      ],
  )
  def hist_kernel(eflat_hbm, out_hbm, hist_v, dbuf_v):
    cid = lax.axis_index("c")
    sid = lax.axis_index("s")
    wid = sid * NC + cid
    base = E + wid * EPW

    def zero_body(i, _):
      hist_v[pl.ds(i * L, L)] = jnp.zeros((L,), jnp.float32)
      return 0
    lax.fori_loop(0, N // L, zero_body, 0)

    ones = jnp.ones((L,), jnp.float32)

    def chunk_body(c, _):
      pltpu.sync_copy(eflat_hbm.at[pl.ds(base + c * CHUNK, CHUNK)], dbuf_v)

      def inner(i, _):
        idx = dbuf_v[pl.ds(i * L, L)]
        plsc.addupdate_scatter(hist_v, [idx], ones)
        return 0
      lax.fori_loop(0, CHUNK // L, inner, 0)
      return 0
    lax.fori_loop(0, NCHUNK, chunk_body, 0)

    pltpu.sync_copy(hist_v, out_hbm.at[wid])

  return hist_kernel


def _make_edge_pass(EPWP, N_pad, D):
  CH = 64
  NBUF = 4
  NCHUNK = EPWP // CH
  RPT = N_pad // NS
  ZR = 64
  CR = 128
  GC = 20
  NG = NCHUNK // GC
  assert RPT % ZR == 0 and RPT % CR == 0
  assert NCHUNK == NG * GC and NG >= 2 and GC > 6

  @functools.partial(
      pl.kernel,
      out_type=jax.ShapeDtypeStruct((NC, N_pad, D), jnp.float32),
      mesh=_mesh(),
      compiler_params=pltpu.CompilerParams(needs_layout_passes=False),
      scratch_types=[
          pltpu.VMEM((2, GC, CH), jnp.int32),
          pltpu.VMEM((2, GC, CH), jnp.int32),
          pltpu.VMEM((NBUF, CH, D), jnp.float32),
          pltpu.VMEM_SHARED((N_pad, D), jnp.float32),
          pltpu.SemaphoreType.DMA,
          pltpu.SemaphoreType.DMA,
          pltpu.SemaphoreType.DMA,
      ],
  )
  def edge_kernel(src_hbm, dst_hbm, xs_hbm, out_hbm,
                  sidx, didx, rows, acc, gsem, isem, ssem):
    cid = lax.axis_index("c")
    sid = lax.axis_index("s")
    wid = sid * NC + cid

    def idx_load(g, gb):
      return (pltpu.make_async_copy(src_hbm.at[wid, g], sidx.at[gb], isem),
              pltpu.make_async_copy(dst_hbm.at[wid, g], didx.at[gb], isem))

    for d in idx_load(0, 0):
      d.start()

    def zrow(r, _):
      def zcol(jj, _):
        rows[0, r, pl.ds(jj * L, L)] = jnp.zeros((L,), jnp.float32)
        return 0
      lax.fori_loop(0, D // L, zcol, 0)
      return 0
    lax.fori_loop(0, ZR, zrow, 0)

    r0 = sid * RPT
    def zacc(k, _):
      pltpu.sync_copy(rows.at[0, pl.ds(0, ZR)],
                      acc.at[pl.ds(r0 + k * ZR, ZR)])
      return 0
    lax.fori_loop(0, RPT // ZR, zacc, 0)

    plsc.subcore_barrier()

    def gwait():
      pltpu.make_async_copy(
          xs_hbm.at[sidx.at[0, 0]], rows.at[0], gsem).wait()

    def swait():
      pltpu.make_async_copy(
          rows.at[0], acc.at[didx.at[0, 0]], ssem).wait()

    def iwait():
      pltpu.make_async_copy(src_hbm.at[wid, 0], sidx.at[0], isem).wait()
      pltpu.make_async_copy(dst_hbm.at[wid, 0], didx.at[0], isem).wait()

    for d in idx_load(0, 0):
      d.wait()
    for d in idx_load(1, 1):
      d.start()
    for j in range(NBUF - 1):
      pltpu.make_async_copy(
          xs_hbm.at[sidx.at[0, j]], rows.at[j], gsem).start()

    def body(j, c):
      k, g, gb, b, b3 = c
      @pl.when(jnp.logical_and(k == 0,
                               jnp.logical_and(j >= GC, g + 1 < NG)))
      def _():
        for d in idx_load(g + 1, 1 - gb):
          d.start()
      @pl.when(jnp.logical_and(k == GC - (NBUF - 1), j + NBUF - 1 < NCHUNK))
      def _():
        iwait()
      k3 = k + NBUF - 1
      cross = k3 >= GC
      k3 = jnp.where(cross, k3 - GC, k3)
      gb3 = jnp.where(cross, 1 - gb, gb)
      @pl.when(j + NBUF - 1 < NCHUNK)
      def _():
        pltpu.make_async_copy(
            xs_hbm.at[sidx.at[gb3, k3]], rows.at[b3], gsem).start()
      gwait()
      pltpu.sync_copy(rows.at[b], acc.at[didx.at[gb, k]], add=True)
      wrap = k + 1 == GC
      kn = jnp.where(wrap, 0, k + 1)
      gn = jnp.where(wrap, g + 1, g)
      gbn = jnp.where(wrap, 1 - gb, gb)
      bn = jnp.where(b + 1 == NBUF, 0, b + 1)
      b3n = jnp.where(b3 + 1 == NBUF, 0, b3 + 1)
      return (kn, gn, gbn, bn, b3n)
    z = jnp.int32(0)
    lax.fori_loop(0, NCHUNK, body,
                  (z, z, z, z, jnp.int32(NBUF - 1)))

    plsc.subcore_barrier()

    def cout(k, _):
      rr = r0 + k * CR
      pltpu.sync_copy(acc.at[pl.ds(rr, CR)], out_hbm.at[cid, pl.ds(rr, CR)])
      return 0
    lax.fori_loop(0, RPT // CR, cout, 0)

  return edge_kernel


def _make_head_gather(N2, TP):
  PPW = TP // NW

  @functools.partial(
      pl.kernel,
      out_type=[jax.ShapeDtypeStruct((TP,), jnp.float32) for _ in range(4)],
      mesh=_mesh(),
      compiler_params=pltpu.CompilerParams(needs_layout_passes=False),
      scratch_types=[
          pltpu.VMEM((N2,), jnp.float32),
          pltpu.VMEM((PPW,), jnp.int32),
          pltpu.VMEM((PPW,), jnp.int32),
          pltpu.VMEM((4, PPW), jnp.float32),
      ],
  )
  def gather_kernel(zy_hbm, t_hbm, c_hbm, y1_hbm, yc0_hbm, y0_hbm, yc1_hbm,
                    zy_v, ti_v, ci_v, ob_v):
    cid = lax.axis_index("c")
    sid = lax.axis_index("s")
    wid = sid * NC + cid
    tb = wid * PPW
    cb = tb

    pltpu.sync_copy(zy_hbm, zy_v)
    pltpu.sync_copy(t_hbm.at[pl.ds(tb, PPW)], ti_v)
    pltpu.sync_copy(c_hbm.at[pl.ds(cb, PPW)], ci_v)

    def body(i, _):
      sl = pl.ds(i * L, L)
      it2 = ti_v[sl] * 2
      ic2 = ci_v[sl] * 2
      ob_v[0, sl] = plsc.load_gather(zy_v, [it2])
      ob_v[1, sl] = plsc.load_gather(zy_v, [it2 + 1])
      ob_v[2, sl] = plsc.load_gather(zy_v, [ic2 + 1])
      ob_v[3, sl] = plsc.load_gather(zy_v, [ic2])
      return 0
    lax.fori_loop(0, PPW // L, body, 0)

    pltpu.sync_copy(ob_v.at[0], y1_hbm.at[pl.ds(tb, PPW)])
    pltpu.sync_copy(ob_v.at[1], yc0_hbm.at[pl.ds(tb, PPW)])
    pltpu.sync_copy(ob_v.at[2], y0_hbm.at[pl.ds(cb, PPW)])
    pltpu.sync_copy(ob_v.at[3], yc1_hbm.at[pl.ds(cb, PPW)])

  return gather_kernel


def _tc0_body(hist_ref, dinv_ref):
  deg = jnp.sum(hist_ref[...], axis=0) + 1.0
  dinv_ref[...] = lax.rsqrt(deg)[:, None]


def _tc1_body(x_ref, w1_ref, dinv_ref, xs1_ref):
  xw = jnp.dot(x_ref[...], w1_ref[...], preferred_element_type=jnp.float32)
  xs1_ref[...] = xw * dinv_ref[...]


def _tc2_body(acc_ref, xs1_ref, dinv_ref, b1_ref, w2_ref, xs2_ref):
  a = acc_ref[0] + acc_ref[1] + xs1_ref[...]
  z1 = a * dinv_ref[...] + b1_ref[...]
  xz1 = jnp.maximum(z1, 0.0)
  xw2 = jnp.dot(xz1, w2_ref[...], preferred_element_type=jnp.float32)
  xs2_ref[...] = xw2 * dinv_ref[...]


def _lrelu(v):
  return jnp.where(v >= 0.0, v, 0.01 * v)


def _tc3_body(acc_ref, xs2_ref, dinv_ref, b2_ref, wy_ref, by_ref,
              wp1_ref, bp1_ref, wp2_ref, bp2_ref,
              xz2_ref, zy_ref, tp_ref):
  a = acc_ref[0] + acc_ref[1] + xs2_ref[...]
  xz2 = a * dinv_ref[...] + b2_ref[...]
  xz2_ref[...] = xz2
  zy = _lrelu(jnp.dot(xz2, wy_ref[...], preferred_element_type=jnp.float32)
              + by_ref[...])
  zy_ref[...] = zy[:, :2]
  h = _lrelu(jnp.dot(xz2, wp1_ref[...], preferred_element_type=jnp.float32)
             + bp1_ref[...])
  tp = _lrelu(jnp.dot(h, wp2_ref[...], preferred_element_type=jnp.float32)
              + bp2_ref[...])
  tp_ref[...] = tp[:, :2]


def kernel(x, edge_index, treat_idx, control_idx, W1, b1, W2, b2,
           Wy1, by1, Wy0, by0, Wp1, bp1, Wp2, bp2):
  N, D = x.shape
  H = W1.shape[1]
  E = edge_index.shape[1]
  T = treat_idx.shape[0]
  C = control_idx.shape[0]
  N_pad = ((N + NW * 128 - 1) // (NW * 128)) * (NW * 128) // 2
  N_pad = ((N + 2047) // 2048) * 2048

  eflat = edge_index.reshape(-1)
  hist32 = _make_hist(E, N_pad)(eflat)

  BC = 1280
  dinv = pl.pallas_call(
      _tc0_body,
      grid=(N_pad // BC,),
      in_specs=[pl.BlockSpec((NW, BC), lambda i: (0, i))],
      out_specs=pl.BlockSpec((BC, 1), lambda i: (i, 0)),
      out_shape=jax.ShapeDtypeStruct((N_pad, 1), jnp.float32),
  )(hist32)

  BR = 1000
  G = N // BR
  xs1 = pl.pallas_call(
      _tc1_body,
      grid=(G,),
      in_specs=[
          pl.BlockSpec((BR, D), lambda i: (i, 0)),
          pl.BlockSpec((D, H), lambda i: (0, 0)),
          pl.BlockSpec((BR, 1), lambda i: (i, 0)),
      ],
      out_specs=pl.BlockSpec((BR, H), lambda i: (i, 0)),
      out_shape=jax.ShapeDtypeStruct((N, H), jnp.float32),
  )(x, W1, dinv)

  EPW = E // NW
  CH = 64
  GC = 20
  EPWP = ((EPW + CH * GC - 1) // (CH * GC)) * (CH * GC)
  srcp = jnp.pad(edge_index[0].reshape(NW, EPW), ((0, 0), (0, EPWP - EPW)),
                 constant_values=0).reshape(NW, EPWP // (CH * GC), GC, CH)
  dstp = jnp.pad(edge_index[1].reshape(NW, EPW), ((0, 0), (0, EPWP - EPW)),
                 constant_values=N_pad - 1).reshape(
                     NW, EPWP // (CH * GC), GC, CH)
  edge_pass = _make_edge_pass(EPWP, N_pad, H)
  acc1 = edge_pass(srcp, dstp, xs1)

  xs2 = pl.pallas_call(
      _tc2_body,
      grid=(G,),
      in_specs=[
          pl.BlockSpec((NC, BR, H), lambda i: (0, i, 0)),
          pl.BlockSpec((BR, H), lambda i: (i, 0)),
          pl.BlockSpec((BR, 1), lambda i: (i, 0)),
          pl.BlockSpec((1, H), lambda i: (0, 0)),
          pl.BlockSpec((H, H), lambda i: (0, 0)),
      ],
      out_specs=pl.BlockSpec((BR, H), lambda i: (i, 0)),
      out_shape=jax.ShapeDtypeStruct((N, H), jnp.float32),
  )(acc1, xs1, dinv, b1[None, :], W2)

  acc2 = edge_pass(srcp, dstp, xs2)

  wy = jnp.concatenate([Wy1, Wy0], axis=1)
  wy_pad = jnp.pad(wy, ((0, 0), (0, H - 2)))
  by_pad = jnp.pad(jnp.concatenate([by1, by0]), (0, H - 2))[None, :]
  wp2_pad = jnp.pad(Wp2, ((0, 0), (0, H - 2)))
  bp2_pad = jnp.pad(bp2, (0, H - 2))[None, :]

  xz2, zy, tprob = pl.pallas_call(
      _tc3_body,
      grid=(G,),
      in_specs=[
          pl.BlockSpec((NC, BR, H), lambda i: (0, i, 0)),
          pl.BlockSpec((BR, H), lambda i: (i, 0)),
          pl.BlockSpec((BR, 1), lambda i: (i, 0)),
          pl.BlockSpec((1, H), lambda i: (0, 0)),
          pl.BlockSpec((H, H), lambda i: (0, 0)),
          pl.BlockSpec((1, H), lambda i: (0, 0)),
          pl.BlockSpec((H, H), lambda i: (0, 0)),
          pl.BlockSpec((1, H), lambda i: (0, 0)),
          pl.BlockSpec((H, H), lambda i: (0, 0)),
          pl.BlockSpec((1, H), lambda i: (0, 0)),
      ],
      out_specs=[
          pl.BlockSpec((BR, H), lambda i: (i, 0)),
          pl.BlockSpec((BR, 2), lambda i: (i, 0)),
          pl.BlockSpec((BR, 2), lambda i: (i, 0)),
      ],
      out_shape=[
          jax.ShapeDtypeStruct((N, H), jnp.float32),
          jax.ShapeDtypeStruct((N_pad, 2), jnp.float32),
          jax.ShapeDtypeStruct((N, 2), jnp.float32),
      ],
  )(acc2, xs2, dinv, b2[None, :], wy_pad, by_pad, Wp1, bp1[None, :],
    wp2_pad, bp2_pad)

  TP = ((max(T, C) + NW * 128 - 1) // (NW * 128)) * (NW * 128)
  tpad = jnp.concatenate([treat_idx, jnp.zeros((TP - T,), jnp.int32)])
  cpad = jnp.concatenate([control_idx, jnp.zeros((TP - C,), jnp.int32)])
  y1p, yc0p, y0p, yc1p = _make_head_gather(2 * N_pad, TP)(
      zy.reshape(-1), tpad, cpad)

  return (y1p[:T], yc0p[:T], y0p[:C], yc1p[:C], tprob, xz2)


# speedup vs baseline: 2.7745x; 2.7745x over previous
import functools
import jax
import jax.numpy as jnp
from jax import lax
from jax.experimental import pallas as pl
from jax.experimental.pallas import tpu as pltpu
from jax.experimental.pallas import tpu_sc as plsc

NC = 2
NS = 16
NW = NC * NS
L = 16

def _mesh():
  return plsc.VectorSubcoreMesh(core_axis_name="c", subcore_axis_name="s",
                                num_cores=NC, num_subcores=NS)


def _make_hist(E, N_pad):
  EPW = E // NW
  CHUNK = 2000
  NCHUNK = EPW // CHUNK
  N = N_pad

  @functools.partial(
      pl.kernel,
      out_type=jax.ShapeDtypeStruct((NW, N), jnp.float32),
      mesh=_mesh(),
      compiler_params=pltpu.CompilerParams(needs_layout_passes=False),
      scratch_types=[
          pltpu.VMEM((N,), jnp.float32),
          pltpu.VMEM((CHUNK,), jnp.int32),
      ],
  )
  def hist_kernel(eflat_hbm, out_hbm, hist_v, dbuf_v):
    cid = lax.axis_index("c")
    sid = lax.axis_index("s")
    wid = sid * NC + cid
    base = E + wid * EPW

    def zero_body(i, _):
      hist_v[pl.ds(i * L, L)] = jnp.zeros((L,), jnp.float32)
      return 0
    lax.fori_loop(0, N // L, zero_body, 0)

    ones = jnp.ones((L,), jnp.float32)

    def chunk_body(c, _):
      pltpu.sync_copy(eflat_hbm.at[pl.ds(base + c * CHUNK, CHUNK)], dbuf_v)

      def inner(i, _):
        idx = dbuf_v[pl.ds(i * L, L)]
        plsc.addupdate_scatter(hist_v, [idx], ones)
        return 0
      lax.fori_loop(0, CHUNK // L, inner, 0)
      return 0
    lax.fori_loop(0, NCHUNK, chunk_body, 0)

    pltpu.sync_copy(hist_v, out_hbm.at[wid])

  return hist_kernel


def _make_edge_pass(E, N_pad, D):
  EPW = E // NW
  CH = 125
  NCHUNK = EPW // CH
  RPT = N_pad // NS
  ZR = 64
  CR = 128
  GC = 16
  NG = NCHUNK // GC
  assert RPT % ZR == 0 and RPT % CR == 0
  assert NCHUNK == NG * GC and GC % 2 == 0

  @functools.partial(
      pl.kernel,
      out_type=jax.ShapeDtypeStruct((NC, N_pad, D), jnp.float32),
      mesh=_mesh(),
      compiler_params=pltpu.CompilerParams(needs_layout_passes=False),
      scratch_types=[
          pltpu.VMEM((2, GC, CH), jnp.int32),
          pltpu.VMEM((2, GC, CH), jnp.int32),
          pltpu.VMEM((2, 128, D), jnp.float32),
          pltpu.VMEM_SHARED((N_pad, D), jnp.float32),
          pltpu.SemaphoreType.DMA,
          pltpu.SemaphoreType.DMA,
      ],
  )
  def edge_kernel(edge_hbm, xs_hbm, out_hbm,
                  sidx, didx, rows, acc, gsem, isem):
    cid = lax.axis_index("c")
    sid = lax.axis_index("s")
    wid = sid * NC + cid

    def load_group(g, gb):
      return (pltpu.make_async_copy(edge_hbm.at[0, wid, g], sidx.at[gb], isem),
              pltpu.make_async_copy(edge_hbm.at[1, wid, g], didx.at[gb], isem))

    for d in load_group(0, 0):
      d.start()

    def zrow(r, _):
      def zcol(j, _):
        rows[0, r, pl.ds(j * L, L)] = jnp.zeros((L,), jnp.float32)
        return 0
      lax.fori_loop(0, D // L, zcol, 0)
      return 0
    lax.fori_loop(0, ZR, zrow, 0)

    r0 = sid * RPT
    def zacc(k, _):
      pltpu.sync_copy(rows.at[0, pl.ds(0, ZR)],
                      acc.at[pl.ds(r0 + k * ZR, ZR)])
      return 0
    lax.fori_loop(0, RPT // ZR, zacc, 0)

    plsc.subcore_barrier()

    for g in range(NG):
      gb = g % 2
      for d in load_group(g, gb):
        d.wait()
      if g + 1 < NG:
        for d in load_group(g + 1, 1 - gb):
          d.start()

      def mkd(j, b):
        return pltpu.make_async_copy(
            xs_hbm.at[sidx.at[gb, j]], rows.at[b, pl.ds(0, CH)], gsem)

      def scat(j, b):
        pltpu.sync_copy(rows.at[b, pl.ds(0, CH)],
                        acc.at[didx.at[gb, j]], add=True)

      mkd(0, 0).start()
      def pair(jj, _):
        j = jj * 2
        mkd(j + 1, 1).start()
        mkd(j, 0).wait()
        scat(j, 0)
        mkd(j + 2, 0).start()
        mkd(j + 1, 1).wait()
        scat(j + 1, 1)
        return 0
      lax.fori_loop(0, GC // 2 - 1, pair, 0)
      mkd(GC - 1, 1).start()
      mkd(GC - 2, 0).wait()
      scat(GC - 2, 0)
      mkd(GC - 1, 1).wait()
      scat(GC - 1, 1)

    plsc.subcore_barrier()

    def cout(k, _):
      rr = r0 + k * CR
      pltpu.sync_copy(acc.at[pl.ds(rr, CR)], out_hbm.at[cid, pl.ds(rr, CR)])
      return 0
    lax.fori_loop(0, RPT // CR, cout, 0)

  return edge_kernel


def _make_head_gather(N2, TP):
  PPW = TP // NW

  @functools.partial(
      pl.kernel,
      out_type=[jax.ShapeDtypeStruct((TP,), jnp.float32) for _ in range(4)],
      mesh=_mesh(),
      compiler_params=pltpu.CompilerParams(needs_layout_passes=False),
      scratch_types=[
          pltpu.VMEM((N2,), jnp.float32),
          pltpu.VMEM((PPW,), jnp.int32),
          pltpu.VMEM((PPW,), jnp.int32),
          pltpu.VMEM((4, PPW), jnp.float32),
      ],
  )
  def gather_kernel(zy_hbm, t_hbm, c_hbm, y1_hbm, yc0_hbm, y0_hbm, yc1_hbm,
                    zy_v, ti_v, ci_v, ob_v):
    cid = lax.axis_index("c")
    sid = lax.axis_index("s")
    wid = sid * NC + cid
    tb = wid * PPW
    cb = tb

    pltpu.sync_copy(zy_hbm, zy_v)
    pltpu.sync_copy(t_hbm.at[pl.ds(tb, PPW)], ti_v)
    pltpu.sync_copy(c_hbm.at[pl.ds(cb, PPW)], ci_v)

    def body(i, _):
      sl = pl.ds(i * L, L)
      it2 = ti_v[sl] * 2
      ic2 = ci_v[sl] * 2
      ob_v[0, sl] = plsc.load_gather(zy_v, [it2])
      ob_v[1, sl] = plsc.load_gather(zy_v, [it2 + 1])
      ob_v[2, sl] = plsc.load_gather(zy_v, [ic2 + 1])
      ob_v[3, sl] = plsc.load_gather(zy_v, [ic2])
      return 0
    lax.fori_loop(0, PPW // L, body, 0)

    pltpu.sync_copy(ob_v.at[0], y1_hbm.at[pl.ds(tb, PPW)])
    pltpu.sync_copy(ob_v.at[1], yc0_hbm.at[pl.ds(tb, PPW)])
    pltpu.sync_copy(ob_v.at[2], y0_hbm.at[pl.ds(cb, PPW)])
    pltpu.sync_copy(ob_v.at[3], yc1_hbm.at[pl.ds(cb, PPW)])

  return gather_kernel


def _tc0_body(hist_ref, dinv_ref):
  deg = jnp.sum(hist_ref[...], axis=0) + 1.0
  dinv_ref[...] = lax.rsqrt(deg)[:, None]


def _tc1_body(x_ref, w1_ref, dinv_ref, xs1_ref):
  xw = jnp.dot(x_ref[...], w1_ref[...], preferred_element_type=jnp.float32)
  xs1_ref[...] = xw * dinv_ref[...]


def _tc2_body(acc_ref, xs1_ref, dinv_ref, b1_ref, w2_ref, xs2_ref):
  a = acc_ref[0] + acc_ref[1] + xs1_ref[...]
  z1 = a * dinv_ref[...] + b1_ref[...]
  xz1 = jnp.maximum(z1, 0.0)
  xw2 = jnp.dot(xz1, w2_ref[...], preferred_element_type=jnp.float32)
  xs2_ref[...] = xw2 * dinv_ref[...]


def _lrelu(v):
  return jnp.where(v >= 0.0, v, 0.01 * v)


def _tc3_body(acc_ref, xs2_ref, dinv_ref, b2_ref, wy_ref, by_ref,
              wp1_ref, bp1_ref, wp2_ref, bp2_ref,
              xz2_ref, zy_ref, tp_ref):
  a = acc_ref[0] + acc_ref[1] + xs2_ref[...]
  xz2 = a * dinv_ref[...] + b2_ref[...]
  xz2_ref[...] = xz2
  zy = _lrelu(jnp.dot(xz2, wy_ref[...], preferred_element_type=jnp.float32)
              + by_ref[...])
  zy_ref[...] = zy[:, :2]
  h = _lrelu(jnp.dot(xz2, wp1_ref[...], preferred_element_type=jnp.float32)
             + bp1_ref[...])
  tp = _lrelu(jnp.dot(h, wp2_ref[...], preferred_element_type=jnp.float32)
              + bp2_ref[...])
  tp_ref[...] = tp[:, :2]


def kernel(x, edge_index, treat_idx, control_idx, W1, b1, W2, b2,
           Wy1, by1, Wy0, by0, Wp1, bp1, Wp2, bp2):
  N, D = x.shape
  H = W1.shape[1]
  E = edge_index.shape[1]
  T = treat_idx.shape[0]
  C = control_idx.shape[0]
  N_pad = ((N + NW * 128 - 1) // (NW * 128)) * (NW * 128) // 2
  N_pad = ((N + 2047) // 2048) * 2048

  eflat = edge_index.reshape(-1)
  hist32 = _make_hist(E, N_pad)(eflat)

  BC = 1280
  dinv = pl.pallas_call(
      _tc0_body,
      grid=(N_pad // BC,),
      in_specs=[pl.BlockSpec((NW, BC), lambda i: (0, i))],
      out_specs=pl.BlockSpec((BC, 1), lambda i: (i, 0)),
      out_shape=jax.ShapeDtypeStruct((N_pad, 1), jnp.float32),
  )(hist32)

  BR = 1000
  G = N // BR
  xs1 = pl.pallas_call(
      _tc1_body,
      grid=(G,),
      in_specs=[
          pl.BlockSpec((BR, D), lambda i: (i, 0)),
          pl.BlockSpec((D, H), lambda i: (0, 0)),
          pl.BlockSpec((BR, 1), lambda i: (i, 0)),
      ],
      out_specs=pl.BlockSpec((BR, H), lambda i: (i, 0)),
      out_shape=jax.ShapeDtypeStruct((N, H), jnp.float32),
  )(x, W1, dinv)

  EPW = E // NW
  CH = 125
  GC = 16
  edge5 = edge_index.reshape(2, NW, EPW // CH // GC, GC, CH)
  edge_pass = _make_edge_pass(E, N_pad, H)
  acc1 = edge_pass(edge5, xs1)

  xs2 = pl.pallas_call(
      _tc2_body,
      grid=(G,),
      in_specs=[
          pl.BlockSpec((NC, BR, H), lambda i: (0, i, 0)),
          pl.BlockSpec((BR, H), lambda i: (i, 0)),
          pl.BlockSpec((BR, 1), lambda i: (i, 0)),
          pl.BlockSpec((1, H), lambda i: (0, 0)),
          pl.BlockSpec((H, H), lambda i: (0, 0)),
      ],
      out_specs=pl.BlockSpec((BR, H), lambda i: (i, 0)),
      out_shape=jax.ShapeDtypeStruct((N, H), jnp.float32),
  )(acc1, xs1, dinv, b1[None, :], W2)

  acc2 = edge_pass(edge5, xs2)

  wy = jnp.concatenate([Wy1, Wy0], axis=1)
  wy_pad = jnp.pad(wy, ((0, 0), (0, H - 2)))
  by_pad = jnp.pad(jnp.concatenate([by1, by0]), (0, H - 2))[None, :]
  wp2_pad = jnp.pad(Wp2, ((0, 0), (0, H - 2)))
  bp2_pad = jnp.pad(bp2, (0, H - 2))[None, :]

  xz2, zy, tprob = pl.pallas_call(
      _tc3_body,
      grid=(G,),
      in_specs=[
          pl.BlockSpec((NC, BR, H), lambda i: (0, i, 0)),
          pl.BlockSpec((BR, H), lambda i: (i, 0)),
          pl.BlockSpec((BR, 1), lambda i: (i, 0)),
          pl.BlockSpec((1, H), lambda i: (0, 0)),
          pl.BlockSpec((H, H), lambda i: (0, 0)),
          pl.BlockSpec((1, H), lambda i: (0, 0)),
          pl.BlockSpec((H, H), lambda i: (0, 0)),
          pl.BlockSpec((1, H), lambda i: (0, 0)),
          pl.BlockSpec((H, H), lambda i: (0, 0)),
          pl.BlockSpec((1, H), lambda i: (0, 0)),
      ],
      out_specs=[
          pl.BlockSpec((BR, H), lambda i: (i, 0)),
          pl.BlockSpec((BR, 2), lambda i: (i, 0)),
          pl.BlockSpec((BR, 2), lambda i: (i, 0)),
      ],
      out_shape=[
          jax.ShapeDtypeStruct((N, H), jnp.float32),
          jax.ShapeDtypeStruct((N_pad, 2), jnp.float32),
          jax.ShapeDtypeStruct((N, 2), jnp.float32),
      ],
  )(acc2, xs2, dinv, b2[None, :], wy_pad, by_pad, Wp1, bp1[None, :],
    wp2_pad, bp2_pad)

  TP = ((max(T, C) + NW * 128 - 1) // (NW * 128)) * (NW * 128)
  tpad = jnp.concatenate([treat_idx, jnp.zeros((TP - T,), jnp.int32)])
  cpad = jnp.concatenate([control_idx, jnp.zeros((TP - C,), jnp.int32)])
  y1p, yc0p, y0p, yc1p = _make_head_gather(2 * N_pad, TP)(
      zy.reshape(-1), tpad, cpad)

  return (y1p[:T], yc0p[:T], y0p[:C], yc1p[:C], tprob, xz2)
